# baseline (speedup 1.0000x reference)
import functools

import jax
import jax.numpy as jnp
from jax import lax
from jax.experimental import pallas as pl
from jax.experimental.pallas import tpu as pltpu
from jax.experimental.pallas import tpu_sc as plsc

N = 50000
N2 = 50048
RPT = N2 // 16
H = 128
CB = 32
NB = 4
NC = 2
NS = 16
BATCH = 128

E = 600000
EPT = 296
EB = EPT * NS
E_PAD = EB * BATCH

EL = 100000
LPT = 25
LB = LPT * NC * NS
EL_PAD = LB * BATCH


@functools.cache
def _mesh():
    return plsc.VectorSubcoreMesh(core_axis_name="c", subcore_axis_name="s",
                                  num_cores=NC, num_subcores=NS)


def _zero_accum_slice(zbuf, accum, base):
    for k in range(15):
        pltpu.sync_copy(zbuf, accum.at[pl.ds(base + k * 200, 200), :])
    pltpu.sync_copy(zbuf.at[pl.ds(0, 128), :],
                    accum.at[pl.ds(base + 3000, 128), :])


G = 8
NCH = EPT // G
NT = 12


def _agg_body(tur0, tur1, tur2, tur3, tru0, tru1, tru2, tru3,
              srcg_h, dsts_h, dstg_h, srcs_h,
              our0, our1, our2, our3, oru0, oru1, oru2, oru3,
              is_a, is_b, is_c, id_a, id_b, id_c, r0, r1, r2, r3,
              zbuf, accum,
              g0, g1, g2, g3, s0, s1, s2, s3, sem_ia, sem_ib, sem_ic):
    c = lax.axis_index("c")
    s = lax.axis_index("s")
    row0 = pl.multiple_of(s * EPT, EPT)
    rows = (r0, r1, r2, r3)
    gsem = (g0, g1, g2, g3)
    ssem = (s0, s1, s2, s3)
    is_x = (is_a, is_b, is_c)
    id_x = (id_a, id_b, id_c)
    isem = (sem_ia, sem_ib, sem_ic)

    def _zfill(j, _):
        zbuf[j, pl.ds(0, 16)] = jnp.zeros((16,), jnp.float32)
        zbuf[j, pl.ds(16, 16)] = jnp.zeros((16,), jnp.float32)
        return 0
    lax.fori_loop(0, zbuf.shape[0], _zfill, 0, unroll=False)

    def do_block(tbl, out, gat_h, sca_h):
        def chunk_copies(ch, is_x, id_x, sem_x):
            r = pl.multiple_of(row0 + ch * G, G)
            return (
                pltpu.make_async_copy(gat_h.at[pl.ds(r, G), :], is_x, sem_x),
                pltpu.make_async_copy(sca_h.at[pl.ds(r, G), :], id_x, sem_x))

        def load_chunk(ch, is_x, id_x, sem_x):
            for d in chunk_copies(ch, is_x, id_x, sem_x):
                d.start()

        def wait_chunk(is_x, id_x, sem_x):
            for d in chunk_copies(0, is_x, id_x, sem_x):
                d.wait()
        base = pl.multiple_of(s * RPT, RPT)
        _zero_accum_slice(zbuf, accum, base)
        plsc.subcore_barrier()

        def start_gather(idx_row, m):
            pltpu.async_copy(tbl.at[idx_row], rows[m], gsem[m])

        def wait_gather(idx_row, m):
            pltpu.make_async_copy(tbl.at[idx_row], rows[m], gsem[m]).wait()

        def start_scatter(m, id_row):
            pltpu.async_copy(rows[m], accum.at[id_row], ssem[m], add=True)

        def wait_scatter(m):
            pltpu.make_async_copy(rows[m], accum.at[id_a.at[0]],
                                  ssem[m]).wait()

        load_chunk(0, is_a, id_a, sem_ia)
        load_chunk(1, is_b, id_b, sem_ib)
        wait_chunk(is_a, id_a, sem_ia)
        for m in range(1, 4):
            pltpu.async_copy(zbuf.at[pl.ds(0, BATCH), :],
                             accum.at[id_a.at[0]], ssem[m], add=True)
        start_gather(is_a.at[0], 0)

        def slot(j, cur_is, cur_id, nxt_is):
            m, m1 = j % 4, (j + 1) % 4
            wait_scatter(m1)
            start_gather(nxt_is, m1)
            wait_gather(cur_is, m)
            start_scatter(m, cur_id)

        def _triple(q, _):
            for j in range(24):
                cur_is = is_x[j // G].at[j % G]
                cur_id = id_x[j // G].at[j % G]
                nj = j + 1
                if nj % G == 0:
                    k = (nj // G) % 3
                    wait_chunk(is_x[k], id_x[k], isem[k])
                nxt_is = is_x[(nj // G) % 3].at[nj % G]
                slot(j, cur_is, cur_id, nxt_is)
                if j == 3:
                    load_chunk(3 * q + 2, is_c, id_c, sem_ic)
                if j == 11:
                    load_chunk(3 * q + 3, is_a, id_a, sem_ia)
                if j == 19:
                    @pl.when(q < NT - 1)
                    def _():
                        load_chunk(3 * q + 4, is_b, id_b, sem_ib)
            return 0
        lax.fori_loop(0, NT, _triple, 0, unroll=False)

        for j in range(G - 1):
            slot(j, is_a.at[j], id_a.at[j], is_a.at[j + 1])
        m = (G - 1) % 4
        wait_gather(is_a.at[G - 1], m)
        start_scatter(m, id_a.at[G - 1])
        for m in range(4):
            wait_scatter(m)
        plsc.subcore_barrier()

        pltpu.sync_copy(accum.at[pl.ds(base, RPT), :],
                        out.at[pl.ds(base, RPT), :])
        plsc.subcore_barrier()

    @pl.when(c == 0)
    def _():
        do_block(tur0, our0, srcg_h, dsts_h)
        do_block(tur1, our1, srcg_h, dsts_h)
        do_block(tru0, oru0, dstg_h, srcs_h)
        do_block(tru1, oru1, dstg_h, srcs_h)

    @pl.when(c == 1)
    def _():
        do_block(tur2, our2, srcg_h, dsts_h)
        do_block(tur3, our3, srcg_h, dsts_h)
        do_block(tru2, oru2, dstg_h, srcs_h)
        do_block(tru3, oru3, dstg_h, srcs_h)


@functools.cache
def _agg():
    return pl.kernel(
        _agg_body,
        out_type=[jax.ShapeDtypeStruct((N2, CB), jnp.float32)
                  for _ in range(2 * NB)],
        mesh=_mesh(),
        compiler_params=pltpu.CompilerParams(use_tc_tiling_on_sc=False),
        scratch_types=[
            pltpu.VMEM((G, BATCH), jnp.int32),
            pltpu.VMEM((G, BATCH), jnp.int32),
            pltpu.VMEM((G, BATCH), jnp.int32),
            pltpu.VMEM((G, BATCH), jnp.int32),
            pltpu.VMEM((G, BATCH), jnp.int32),
            pltpu.VMEM((G, BATCH), jnp.int32),
            pltpu.VMEM((BATCH, CB), jnp.float32),
            pltpu.VMEM((BATCH, CB), jnp.float32),
            pltpu.VMEM((BATCH, CB), jnp.float32),
            pltpu.VMEM((BATCH, CB), jnp.float32),
            pltpu.VMEM((200, CB), jnp.float32),
            pltpu.VMEM_SHARED((N2, CB), jnp.float32),
        ] + [pltpu.SemaphoreType.DMA] * 11,
    )


def _cnt_body(dst_h, src_h, cnt_r, cnt_u, idx, ones, zbuf, accum, sem):
    c = lax.axis_index("c")
    s = lax.axis_index("s")

    def _ofill(j, _):
        ones[j, pl.ds(0, 16)] = jnp.ones((16,), jnp.float32)
        return 0
    lax.fori_loop(0, BATCH, _ofill, 0, unroll=False)

    def _zfill(j, _):
        zbuf[j, pl.ds(0, 16)] = jnp.zeros((16,), jnp.float32)
        return 0
    lax.fori_loop(0, zbuf.shape[0], _zfill, 0, unroll=False)

    def do_count(idx_h, out):
        row0 = pl.multiple_of(s * EPT, EPT)
        pltpu.sync_copy(idx_h.at[pl.ds(row0, EPT), :], idx)
        base = pl.multiple_of(s * RPT, RPT)
        _zero_accum_slice(zbuf, accum, base)
        plsc.subcore_barrier()

        def _q(iq, _):
            i0 = iq * 4
            for j in range(4):
                pltpu.async_copy(ones, accum.at[idx.at[i0 + j]], sem,
                                 add=True)
            for j in range(4):
                pltpu.make_async_copy(ones, accum.at[idx.at[i0 + j]],
                                      sem).wait()
            return 0
        lax.fori_loop(0, EPT // 4, _q, 0, unroll=False)
        plsc.subcore_barrier()

        pltpu.sync_copy(accum.at[pl.ds(base, RPT), :],
                        out.at[pl.ds(base, RPT), :])
        plsc.subcore_barrier()

    @pl.when(c == 0)
    def _():
        do_count(dst_h, cnt_r)

    @pl.when(c == 1)
    def _():
        do_count(src_h, cnt_u)


@functools.cache
def _cnt():
    return pl.kernel(
        _cnt_body,
        out_type=[jax.ShapeDtypeStruct((N2, 16), jnp.float32)
                  for _ in range(2)],
        mesh=_mesh(),
        compiler_params=pltpu.CompilerParams(use_tc_tiling_on_sc=False),
        scratch_types=[
            pltpu.VMEM((EPT, BATCH), jnp.int32),
            pltpu.VMEM((BATCH, 16), jnp.float32),
            pltpu.VMEM((200, 16), jnp.float32),
            pltpu.VMEM_SHARED((N2, 16), jnp.float32),
            pltpu.SemaphoreType.DMA,
        ],
    )


def _cls_body(u0, u1, u2, u3, r0, r1, r2, r3, eliu_h, elir_h,
              fu0, fu1, fu2, fu3, fr0, fr1, fr2, fr3,
              idxu, idxr, gb0, gb1, gb2, gb3, gb4, gb5, gb6, gb7,
              sem_g, sem_w):
    c = lax.axis_index("c")
    s = lax.axis_index("s")
    wid = s * NC + c
    tabs = (u0, u1, u2, u3, r0, r1, r2, r3)
    outs = (fu0, fu1, fu2, fu3, fr0, fr1, fr2, fr3)
    bufs = (gb0, gb1, gb2, gb3, gb4, gb5, gb6, gb7)

    e0 = pl.multiple_of(wid * LPT * BATCH, BATCH)
    pltpu.sync_copy(eliu_h.at[pl.ds(e0, LPT * BATCH)], idxu)
    pltpu.sync_copy(elir_h.at[pl.ds(e0, LPT * BATCH)], idxr)

    def _batch(i, _):
        iu = idxu.at[pl.ds(i * BATCH, BATCH)]
        ir = idxr.at[pl.ds(i * BATCH, BATCH)]
        idxs = (iu, iu, iu, iu, ir, ir, ir, ir)
        for k in range(8):
            pltpu.async_copy(tabs[k].at[idxs[k]], bufs[k], sem_g)
        off = e0 + i * BATCH
        for k in range(8):
            pltpu.make_async_copy(tabs[k].at[idxs[k]], bufs[k], sem_g).wait()
            pltpu.async_copy(bufs[k], outs[k].at[pl.ds(off, BATCH), :],
                             sem_w)
        for k in range(8):
            pltpu.make_async_copy(bufs[k], outs[k].at[pl.ds(off, BATCH), :],
                                  sem_w).wait()
        return 0
    lax.fori_loop(0, LPT, _batch, 0, unroll=False)


@functools.cache
def _cls():
    return pl.kernel(
        _cls_body,
        out_type=[jax.ShapeDtypeStruct((EL_PAD, CB), jnp.float32)
                  for _ in range(8)],
        mesh=_mesh(),
        compiler_params=pltpu.CompilerParams(use_tc_tiling_on_sc=False),
        scratch_types=(
            [pltpu.VMEM((LPT * BATCH,), jnp.int32) for _ in range(2)]
            + [pltpu.VMEM((BATCH, CB), jnp.float32) for _ in range(8)]
            + [pltpu.SemaphoreType.DMA, pltpu.SemaphoreType.DMA]
        ),
    )


def _dot_tc(fu0, fu1, fu2, fu3, fr0, fr1, fr2, fr3, o_ref):
    fu = jnp.concatenate([fu0[...], fu1[...], fu2[...], fu3[...]], axis=1)
    fr = jnp.concatenate([fr0[...], fr1[...], fr2[...], fr3[...]], axis=1)
    o_ref[...] = jnp.sum(fu * fr, axis=1, keepdims=True)


_RD = 2048


def _dot_call(feats):
    return pl.pallas_call(
        _dot_tc,
        grid=(EL_PAD // _RD,),
        in_specs=[pl.BlockSpec((_RD, CB), lambda i: (i, 0))
                  for _ in range(8)],
        out_specs=pl.BlockSpec((_RD, 1), lambda i: (i, 0)),
        out_shape=jax.ShapeDtypeStruct((EL_PAD, 1), jnp.float32),
    )(*feats)


_R = 1000


def _init_tc(x_ref, w_ref, b_ref, e_ref, o0, o1, o2, o3):
    x = jnp.dot(x_ref[...], w_ref[...],
                preferred_element_type=jnp.float32)
    x = x + b_ref[...] + e_ref[...]
    for b, o in enumerate((o0, o1, o2, o3)):
        o[...] = x[:, b * CB:(b + 1) * CB]


def _init_call(x, w, bias, emb):
    k = x.shape[1]
    return pl.pallas_call(
        _init_tc,
        grid=(N // _R,),
        in_specs=[
            pl.BlockSpec((_R, k), lambda i: (i, 0)),
            pl.BlockSpec((k, H), lambda i: (0, 0)),
            pl.BlockSpec((1, H), lambda i: (0, 0)),
            pl.BlockSpec((_R, H), lambda i: (i, 0)),
        ],
        out_specs=[pl.BlockSpec((_R, CB), lambda i: (i, 0))
                   for _ in range(NB)],
        out_shape=[jax.ShapeDtypeStruct((N2, CB), jnp.float32)
                   for _ in range(NB)],
    )(x, w, bias.reshape(1, H), emb)


def _sage_one(relu, ablks, c_ref, xblks, wl_ref, bl_ref, wr_ref):
    agg = jnp.concatenate([a[...] for a in ablks], axis=1)
    xd = jnp.concatenate([x[...] for x in xblks], axis=1)
    cnt = c_ref[:, 0:1]
    recip = 1.0 / jnp.maximum(cnt, 1.0)
    out = recip * jnp.dot(agg, wl_ref[...],
                          preferred_element_type=jnp.float32)
    out = out + bl_ref[...]
    out = out + jnp.dot(xd, wr_ref[...], preferred_element_type=jnp.float32)
    if relu:
        out = jnp.maximum(out, 0.0)
    return out


def _sage_call(agg, cnt, xdst, wl, bl, wr, relu):
    def body(*refs):
        outs = refs[12:]
        out = _sage_one(relu, refs[0:4], refs[4], refs[5:9], refs[9],
                        refs[10], refs[11])
        for b in range(NB):
            outs[b][...] = out[:, b * CB:(b + 1) * CB]

    blk = pl.BlockSpec((_R, CB), lambda i: (i, 0))
    cntblk = pl.BlockSpec((_R, 16), lambda i: (i, 0))
    wblk = pl.BlockSpec((H, H), lambda i: (0, 0))
    bblk = pl.BlockSpec((1, H), lambda i: (0, 0))
    return pl.pallas_call(
        body,
        grid=(N // _R,),
        in_specs=[blk] * 4 + [cntblk] + [blk] * 4 + [wblk, bblk, wblk],
        out_specs=[blk for _ in range(NB)],
        out_shape=[jax.ShapeDtypeStruct((N2, CB), jnp.float32)
                   for _ in range(NB)],
    )(*agg, cnt, *xdst, wl, bl.reshape(1, H), wr)


def kernel(x_user, x_restaurant, user_node_id, restaurant_node_id,
           edge_index, edge_label_index, W_user, b_user, W_rest, b_rest,
           emb_user, emb_rest,
           c1ur_Wl, c1ur_bl, c1ur_Wr, c1ru_Wl, c1ru_bl, c1ru_Wr,
           c2ur_Wl, c2ur_bl, c2ur_Wr, c2ru_Wl, c2ru_bl, c2ru_Wr):
    i32 = jnp.int32
    src, dst = edge_index[0], edge_index[1]
    zpad = jnp.zeros((E_PAD - E,), i32)
    dpad = jnp.full((E_PAD - E,), N, i32)
    src_g = jnp.concatenate([src, zpad]).reshape(EB, BATCH)
    src_s = jnp.concatenate([src, dpad]).reshape(EB, BATCH)
    dst_g = jnp.concatenate([dst, zpad]).reshape(EB, BATCH)
    dst_s = jnp.concatenate([dst, dpad]).reshape(EB, BATCH)
    lpad = jnp.zeros((EL_PAD - EL,), i32)
    eliu = jnp.concatenate([edge_label_index[0], lpad])
    elir = jnp.concatenate([edge_label_index[1], lpad])

    cnt_r, cnt_u = _cnt()(dst_s, src_s)

    xu = _init_call(x_user, W_user, b_user, emb_user)
    xr = _init_call(x_restaurant, W_rest, b_rest, emb_rest)

    aggs = _agg()(*xu, *xr, src_g, dst_s, dst_g, src_s)
    xr1 = _sage_call(aggs[:NB], cnt_r, xr, c1ur_Wl, c1ur_bl, c1ur_Wr, True)
    xu1 = _sage_call(aggs[NB:], cnt_u, xu, c1ru_Wl, c1ru_bl, c1ru_Wr, True)
    aggs = _agg()(*xu1, *xr1, src_g, dst_s, dst_g, src_s)
    xr2 = _sage_call(aggs[:NB], cnt_r, xr1, c2ur_Wl, c2ur_bl, c2ur_Wr, True)
    xu2 = _sage_call(aggs[NB:], cnt_u, xu1, c2ru_Wl, c2ru_bl, c2ru_Wr, True)
    aggs = _agg()(*xu2, *xr2, src_g, dst_s, dst_g, src_s)
    xr3 = _sage_call(aggs[:NB], cnt_r, xr2, c2ur_Wl, c2ur_bl, c2ur_Wr, False)
    xu3 = _sage_call(aggs[NB:], cnt_u, xu2, c2ru_Wl, c2ru_bl, c2ru_Wr, False)

    feats = _cls()(*xu3, *xr3, eliu, elir)
    scores = _dot_call(feats)
    return scores[:EL, 0]


# speedup vs baseline: 1.2244x; 1.2244x over previous
import functools

import jax
import jax.numpy as jnp
from jax import lax
from jax.experimental import pallas as pl
from jax.experimental.pallas import tpu as pltpu
from jax.experimental.pallas import tpu_sc as plsc

N = 50000
N2 = 50048
RPT = N2 // 16
H = 128
CB = 32
NB = 4
NC = 2
NS = 16
BATCH = 128

E = 600000
EPT = 296
EB = EPT * NS
E_PAD = EB * BATCH

EL = 100000
LPT = 25
LB = LPT * NC * NS
EL_PAD = LB * BATCH


@functools.cache
def _mesh():
    return plsc.VectorSubcoreMesh(core_axis_name="c", subcore_axis_name="s",
                                  num_cores=NC, num_subcores=NS)


def _zero_accum_slice(zbuf, accum, base):
    for k in range(15):
        pltpu.sync_copy(zbuf, accum.at[pl.ds(base + k * 200, 200), :])
    pltpu.sync_copy(zbuf.at[pl.ds(0, 128), :],
                    accum.at[pl.ds(base + 3000, 128), :])


G = 8
NCH = EPT // G
NT = 12


def _agg_body(t0, t1, t2, t3, gat_h, sca_h, o0, o1, o2, o3,
              is_a, is_b, is_c, id_a, id_b, id_c, r0, r1, r2, r3,
              zbuf, accum,
              g0, g1, g2, g3, s0, s1, s2, s3, sem_ia, sem_ib, sem_ic):
    c = lax.axis_index("c")
    s = lax.axis_index("s")
    row0 = pl.multiple_of(s * EPT, EPT)
    rows = (r0, r1, r2, r3)
    gsem = (g0, g1, g2, g3)
    ssem = (s0, s1, s2, s3)
    is_x = (is_a, is_b, is_c)
    id_x = (id_a, id_b, id_c)
    isem = (sem_ia, sem_ib, sem_ic)

    def _zfill(j, _):
        zbuf[j, pl.ds(0, 16)] = jnp.zeros((16,), jnp.float32)
        zbuf[j, pl.ds(16, 16)] = jnp.zeros((16,), jnp.float32)
        return 0
    lax.fori_loop(0, zbuf.shape[0], _zfill, 0, unroll=False)

    def do_block(tbl, out):
        def chunk_copies(ch, is_x, id_x, sem_x):
            r = pl.multiple_of(row0 + ch * G, G)
            return (
                pltpu.make_async_copy(gat_h.at[pl.ds(r, G), :], is_x, sem_x),
                pltpu.make_async_copy(sca_h.at[pl.ds(r, G), :], id_x, sem_x))

        def load_chunk(ch, is_x, id_x, sem_x):
            for d in chunk_copies(ch, is_x, id_x, sem_x):
                d.start()

        def wait_chunk(is_x, id_x, sem_x):
            for d in chunk_copies(0, is_x, id_x, sem_x):
                d.wait()
        base = pl.multiple_of(s * RPT, RPT)
        _zero_accum_slice(zbuf, accum, base)
        plsc.subcore_barrier()

        def start_gather(idx_row, m):
            pltpu.async_copy(tbl.at[idx_row], rows[m], gsem[m])

        def wait_gather(idx_row, m):
            pltpu.make_async_copy(tbl.at[idx_row], rows[m], gsem[m]).wait()

        def start_scatter(m, id_row):
            pltpu.async_copy(rows[m], accum.at[id_row], ssem[m], add=True)

        def wait_scatter(m):
            pltpu.make_async_copy(rows[m], accum.at[id_a.at[0]],
                                  ssem[m]).wait()

        load_chunk(0, is_a, id_a, sem_ia)
        load_chunk(1, is_b, id_b, sem_ib)
        wait_chunk(is_a, id_a, sem_ia)
        for m in range(1, 4):
            pltpu.async_copy(zbuf.at[pl.ds(0, BATCH), :],
                             accum.at[id_a.at[0]], ssem[m], add=True)
        start_gather(is_a.at[0], 0)

        def slot(j, cur_is, cur_id, nxt_is):
            m, m1 = j % 4, (j + 1) % 4
            wait_scatter(m1)
            start_gather(nxt_is, m1)
            wait_gather(cur_is, m)
            start_scatter(m, cur_id)

        def _triple(q, _):
            for j in range(24):
                cur_is = is_x[j // G].at[j % G]
                cur_id = id_x[j // G].at[j % G]
                nj = j + 1
                if nj % G == 0:
                    k = (nj // G) % 3
                    wait_chunk(is_x[k], id_x[k], isem[k])
                nxt_is = is_x[(nj // G) % 3].at[nj % G]
                slot(j, cur_is, cur_id, nxt_is)
                if j == 3:
                    load_chunk(3 * q + 2, is_c, id_c, sem_ic)
                if j == 11:
                    load_chunk(3 * q + 3, is_a, id_a, sem_ia)
                if j == 19:
                    @pl.when(q < NT - 1)
                    def _():
                        load_chunk(3 * q + 4, is_b, id_b, sem_ib)
            return 0
        lax.fori_loop(0, NT, _triple, 0, unroll=False)

        for j in range(G - 1):
            slot(j, is_a.at[j], id_a.at[j], is_a.at[j + 1])
        m = (G - 1) % 4
        wait_gather(is_a.at[G - 1], m)
        start_scatter(m, id_a.at[G - 1])
        for m in range(4):
            wait_scatter(m)
        plsc.subcore_barrier()

        pltpu.sync_copy(accum.at[pl.ds(base, RPT), :],
                        out.at[pl.ds(base, RPT), :])
        plsc.subcore_barrier()

    @pl.when(c == 0)
    def _():
        do_block(t0, o0)
        do_block(t1, o1)

    @pl.when(c == 1)
    def _():
        do_block(t2, o2)
        do_block(t3, o3)


@functools.cache
def _agg():
    return pl.kernel(
        _agg_body,
        out_type=[jax.ShapeDtypeStruct((N2, CB), jnp.float32)
                  for _ in range(NB)],
        mesh=_mesh(),
        compiler_params=pltpu.CompilerParams(use_tc_tiling_on_sc=False),
        scratch_types=[
            pltpu.VMEM((G, BATCH), jnp.int32),
            pltpu.VMEM((G, BATCH), jnp.int32),
            pltpu.VMEM((G, BATCH), jnp.int32),
            pltpu.VMEM((G, BATCH), jnp.int32),
            pltpu.VMEM((G, BATCH), jnp.int32),
            pltpu.VMEM((G, BATCH), jnp.int32),
            pltpu.VMEM((BATCH, CB), jnp.float32),
            pltpu.VMEM((BATCH, CB), jnp.float32),
            pltpu.VMEM((BATCH, CB), jnp.float32),
            pltpu.VMEM((BATCH, CB), jnp.float32),
            pltpu.VMEM((200, CB), jnp.float32),
            pltpu.VMEM_SHARED((N2, CB), jnp.float32),
        ] + [pltpu.SemaphoreType.DMA] * 11,
    )


def _cnt_body(dst_h, src_h, cnt_r, cnt_u, idx, ones, zbuf, accum, sem):
    c = lax.axis_index("c")
    s = lax.axis_index("s")

    def _ofill(j, _):
        ones[j, pl.ds(0, 16)] = jnp.ones((16,), jnp.float32)
        return 0
    lax.fori_loop(0, BATCH, _ofill, 0, unroll=False)

    def _zfill(j, _):
        zbuf[j, pl.ds(0, 16)] = jnp.zeros((16,), jnp.float32)
        return 0
    lax.fori_loop(0, zbuf.shape[0], _zfill, 0, unroll=False)

    def do_count(idx_h, out):
        row0 = pl.multiple_of(s * EPT, EPT)
        pltpu.sync_copy(idx_h.at[pl.ds(row0, EPT), :], idx)
        base = pl.multiple_of(s * RPT, RPT)
        _zero_accum_slice(zbuf, accum, base)
        plsc.subcore_barrier()

        def _q(iq, _):
            i0 = iq * 4
            for j in range(4):
                pltpu.async_copy(ones, accum.at[idx.at[i0 + j]], sem,
                                 add=True)
            for j in range(4):
                pltpu.make_async_copy(ones, accum.at[idx.at[i0 + j]],
                                      sem).wait()
            return 0
        lax.fori_loop(0, EPT // 4, _q, 0, unroll=False)
        plsc.subcore_barrier()

        pltpu.sync_copy(accum.at[pl.ds(base, RPT), :],
                        out.at[pl.ds(base, RPT), :])
        plsc.subcore_barrier()

    @pl.when(c == 0)
    def _():
        do_count(dst_h, cnt_r)

    @pl.when(c == 1)
    def _():
        do_count(src_h, cnt_u)


@functools.cache
def _cnt():
    return pl.kernel(
        _cnt_body,
        out_type=[jax.ShapeDtypeStruct((N2, 16), jnp.float32)
                  for _ in range(2)],
        mesh=_mesh(),
        compiler_params=pltpu.CompilerParams(use_tc_tiling_on_sc=False),
        scratch_types=[
            pltpu.VMEM((EPT, BATCH), jnp.int32),
            pltpu.VMEM((BATCH, 16), jnp.float32),
            pltpu.VMEM((200, 16), jnp.float32),
            pltpu.VMEM_SHARED((N2, 16), jnp.float32),
            pltpu.SemaphoreType.DMA,
        ],
    )


def _cls_body(u0, u1, u2, u3, r0, r1, r2, r3, eliu_h, elir_h,
              fu0, fu1, fu2, fu3, fr0, fr1, fr2, fr3,
              idxu, idxr, gb0, gb1, gb2, gb3, gb4, gb5, gb6, gb7,
              sem_g, sem_w):
    c = lax.axis_index("c")
    s = lax.axis_index("s")
    wid = s * NC + c
    tabs = (u0, u1, u2, u3, r0, r1, r2, r3)
    outs = (fu0, fu1, fu2, fu3, fr0, fr1, fr2, fr3)
    bufs = (gb0, gb1, gb2, gb3, gb4, gb5, gb6, gb7)

    e0 = pl.multiple_of(wid * LPT * BATCH, BATCH)
    pltpu.sync_copy(eliu_h.at[pl.ds(e0, LPT * BATCH)], idxu)
    pltpu.sync_copy(elir_h.at[pl.ds(e0, LPT * BATCH)], idxr)

    def _batch(i, _):
        iu = idxu.at[pl.ds(i * BATCH, BATCH)]
        ir = idxr.at[pl.ds(i * BATCH, BATCH)]
        idxs = (iu, iu, iu, iu, ir, ir, ir, ir)
        for k in range(8):
            pltpu.async_copy(tabs[k].at[idxs[k]], bufs[k], sem_g)
        off = e0 + i * BATCH
        for k in range(8):
            pltpu.make_async_copy(tabs[k].at[idxs[k]], bufs[k], sem_g).wait()
            pltpu.async_copy(bufs[k], outs[k].at[pl.ds(off, BATCH), :],
                             sem_w)
        for k in range(8):
            pltpu.make_async_copy(bufs[k], outs[k].at[pl.ds(off, BATCH), :],
                                  sem_w).wait()
        return 0
    lax.fori_loop(0, LPT, _batch, 0, unroll=False)


@functools.cache
def _cls():
    return pl.kernel(
        _cls_body,
        out_type=[jax.ShapeDtypeStruct((EL_PAD, CB), jnp.float32)
                  for _ in range(8)],
        mesh=_mesh(),
        compiler_params=pltpu.CompilerParams(use_tc_tiling_on_sc=False),
        scratch_types=(
            [pltpu.VMEM((LPT * BATCH,), jnp.int32) for _ in range(2)]
            + [pltpu.VMEM((BATCH, CB), jnp.float32) for _ in range(8)]
            + [pltpu.SemaphoreType.DMA, pltpu.SemaphoreType.DMA]
        ),
    )


def _dot_tc(fu0, fu1, fu2, fu3, fr0, fr1, fr2, fr3, o_ref):
    fu = jnp.concatenate([fu0[...], fu1[...], fu2[...], fu3[...]], axis=1)
    fr = jnp.concatenate([fr0[...], fr1[...], fr2[...], fr3[...]], axis=1)
    o_ref[...] = jnp.sum(fu * fr, axis=1, keepdims=True)


_RD = 2048


def _dot_call(feats):
    return pl.pallas_call(
        _dot_tc,
        grid=(EL_PAD // _RD,),
        in_specs=[pl.BlockSpec((_RD, CB), lambda i: (i, 0))
                  for _ in range(8)],
        out_specs=pl.BlockSpec((_RD, 1), lambda i: (i, 0)),
        out_shape=jax.ShapeDtypeStruct((EL_PAD, 1), jnp.float32),
    )(*feats)


_R = 1000


def _init_tc(x_ref, w_ref, b_ref, e_ref, o0, o1, o2, o3):
    x = jnp.dot(x_ref[...], w_ref[...],
                preferred_element_type=jnp.float32)
    x = x + b_ref[...] + e_ref[...]
    for b, o in enumerate((o0, o1, o2, o3)):
        o[...] = x[:, b * CB:(b + 1) * CB]


def _init_call(x, w, bias, emb):
    k = x.shape[1]
    return pl.pallas_call(
        _init_tc,
        grid=(N // _R,),
        in_specs=[
            pl.BlockSpec((_R, k), lambda i: (i, 0)),
            pl.BlockSpec((k, H), lambda i: (0, 0)),
            pl.BlockSpec((1, H), lambda i: (0, 0)),
            pl.BlockSpec((_R, H), lambda i: (i, 0)),
        ],
        out_specs=[pl.BlockSpec((_R, CB), lambda i: (i, 0))
                   for _ in range(NB)],
        out_shape=[jax.ShapeDtypeStruct((N2, CB), jnp.float32)
                   for _ in range(NB)],
    )(x, w, bias.reshape(1, H), emb)


def _sage_one(relu, ablks, c_ref, xblks, wl_ref, bl_ref, wr_ref):
    agg = jnp.concatenate([a[...] for a in ablks], axis=1)
    xd = jnp.concatenate([x[...] for x in xblks], axis=1)
    cnt = c_ref[:, 0:1]
    recip = 1.0 / jnp.maximum(cnt, 1.0)
    out = recip * jnp.dot(agg, wl_ref[...],
                          preferred_element_type=jnp.float32)
    out = out + bl_ref[...]
    out = out + jnp.dot(xd, wr_ref[...], preferred_element_type=jnp.float32)
    if relu:
        out = jnp.maximum(out, 0.0)
    return out


def _sage_call(agg, cnt, xdst, wl, bl, wr, relu):
    def body(*refs):
        outs = refs[12:]
        out = _sage_one(relu, refs[0:4], refs[4], refs[5:9], refs[9],
                        refs[10], refs[11])
        for b in range(NB):
            outs[b][...] = out[:, b * CB:(b + 1) * CB]

    blk = pl.BlockSpec((_R, CB), lambda i: (i, 0))
    cntblk = pl.BlockSpec((_R, 16), lambda i: (i, 0))
    wblk = pl.BlockSpec((H, H), lambda i: (0, 0))
    bblk = pl.BlockSpec((1, H), lambda i: (0, 0))
    return pl.pallas_call(
        body,
        grid=(N // _R,),
        in_specs=[blk] * 4 + [cntblk] + [blk] * 4 + [wblk, bblk, wblk],
        out_specs=[blk for _ in range(NB)],
        out_shape=[jax.ShapeDtypeStruct((N2, CB), jnp.float32)
                   for _ in range(NB)],
    )(*agg, cnt, *xdst, wl, bl.reshape(1, H), wr)


def kernel(x_user, x_restaurant, user_node_id, restaurant_node_id,
           edge_index, edge_label_index, W_user, b_user, W_rest, b_rest,
           emb_user, emb_rest,
           c1ur_Wl, c1ur_bl, c1ur_Wr, c1ru_Wl, c1ru_bl, c1ru_Wr,
           c2ur_Wl, c2ur_bl, c2ur_Wr, c2ru_Wl, c2ru_bl, c2ru_Wr):
    i32 = jnp.int32
    src, dst = edge_index[0], edge_index[1]
    zpad = jnp.zeros((E_PAD - E,), i32)
    dpad = jnp.full((E_PAD - E,), N, i32)
    src_g = jnp.concatenate([src, zpad]).reshape(EB, BATCH)
    src_s = jnp.concatenate([src, dpad]).reshape(EB, BATCH)
    dst_g = jnp.concatenate([dst, zpad]).reshape(EB, BATCH)
    dst_s = jnp.concatenate([dst, dpad]).reshape(EB, BATCH)
    lpad = jnp.zeros((EL_PAD - EL,), i32)
    eliu = jnp.concatenate([edge_label_index[0], lpad])
    elir = jnp.concatenate([edge_label_index[1], lpad])

    cnt_r, cnt_u = _cnt()(dst_s, src_s)

    xu = _init_call(x_user, W_user, b_user, emb_user)
    xr = _init_call(x_restaurant, W_rest, b_rest, emb_rest)

    agg_r = _agg()(*xu, src_g, dst_s)
    agg_u = _agg()(*xr, dst_g, src_s)
    xr1 = _sage_call(agg_r, cnt_r, xr, c1ur_Wl, c1ur_bl, c1ur_Wr, True)
    xu1 = _sage_call(agg_u, cnt_u, xu, c1ru_Wl, c1ru_bl, c1ru_Wr, True)
    agg_r = _agg()(*xu1, src_g, dst_s)
    agg_u = _agg()(*xr1, dst_g, src_s)
    xr2 = _sage_call(agg_r, cnt_r, xr1, c2ur_Wl, c2ur_bl, c2ur_Wr, True)
    xu2 = _sage_call(agg_u, cnt_u, xu1, c2ru_Wl, c2ru_bl, c2ru_Wr, True)
    agg_r = _agg()(*xu2, src_g, dst_s)
    agg_u = _agg()(*xr2, dst_g, src_s)
    xr3 = _sage_call(agg_r, cnt_r, xr2, c2ur_Wl, c2ur_bl, c2ur_Wr, False)
    xu3 = _sage_call(agg_u, cnt_u, xu2, c2ru_Wl, c2ru_bl, c2ru_Wr, False)

    feats = _cls()(*xu3, *xr3, eliu, elir)
    scores = _dot_call(feats)
    return scores[:EL, 0]


# speedup vs baseline: 1.2329x; 1.0070x over previous
import functools

import jax
import jax.numpy as jnp
from jax import lax
from jax.experimental import pallas as pl
from jax.experimental.pallas import tpu as pltpu
from jax.experimental.pallas import tpu_sc as plsc

N = 50000
N2 = 50048
RPT = N2 // 16
H = 128
CB = 32
NB = 4
NC = 2
NS = 16
BATCH = 128

E = 600000
EPT = 296
EB = EPT * NS
E_PAD = EB * BATCH

EL = 100000
LPT = 25
LB = LPT * NC * NS
EL_PAD = LB * BATCH


@functools.cache
def _mesh():
    return plsc.VectorSubcoreMesh(core_axis_name="c", subcore_axis_name="s",
                                  num_cores=NC, num_subcores=NS)


def _zero_accum_slice(zbuf, accum, base):
    for k in range(15):
        pltpu.sync_copy(zbuf, accum.at[pl.ds(base + k * 200, 200), :])
    pltpu.sync_copy(zbuf.at[pl.ds(0, 128), :],
                    accum.at[pl.ds(base + 3000, 128), :])


G = 8
NCH = EPT // G
NT = 12


def _agg_body(t0, t2, gat_h, sca_h, o0, o2,
              is_a, is_b, is_c, id_a, id_b, id_c, r0, r1, r2, r3,
              zbuf, accum,
              g0, g1, g2, g3, s0, s1, s2, s3, sem_ia, sem_ib, sem_ic):
    c = lax.axis_index("c")
    s = lax.axis_index("s")
    row0 = pl.multiple_of(s * EPT, EPT)
    rows = (r0, r1, r2, r3)
    gsem = (g0, g1, g2, g3)
    ssem = (s0, s1, s2, s3)
    is_x = (is_a, is_b, is_c)
    id_x = (id_a, id_b, id_c)
    isem = (sem_ia, sem_ib, sem_ic)

    def _zfill(j, _):
        zbuf[j, pl.ds(0, 16)] = jnp.zeros((16,), jnp.float32)
        zbuf[j, pl.ds(16, 16)] = jnp.zeros((16,), jnp.float32)
        return 0
    lax.fori_loop(0, zbuf.shape[0], _zfill, 0, unroll=False)

    def do_block(tbl, out):
        def chunk_copies(ch, is_x, id_x, sem_x):
            r = pl.multiple_of(row0 + ch * G, G)
            return (
                pltpu.make_async_copy(gat_h.at[pl.ds(r, G), :], is_x, sem_x),
                pltpu.make_async_copy(sca_h.at[pl.ds(r, G), :], id_x, sem_x))

        def load_chunk(ch, is_x, id_x, sem_x):
            for d in chunk_copies(ch, is_x, id_x, sem_x):
                d.start()

        def wait_chunk(is_x, id_x, sem_x):
            for d in chunk_copies(0, is_x, id_x, sem_x):
                d.wait()
        base = pl.multiple_of(s * RPT, RPT)
        _zero_accum_slice(zbuf, accum, base)
        plsc.subcore_barrier()

        def start_gather(idx_row, m):
            pltpu.async_copy(tbl.at[idx_row], rows[m], gsem[m])

        def wait_gather(idx_row, m):
            pltpu.make_async_copy(tbl.at[idx_row], rows[m], gsem[m]).wait()

        def start_scatter(m, id_row):
            pltpu.async_copy(rows[m], accum.at[id_row], ssem[m], add=True)

        def wait_scatter(m):
            pltpu.make_async_copy(rows[m], accum.at[id_a.at[0]],
                                  ssem[m]).wait()

        load_chunk(0, is_a, id_a, sem_ia)
        load_chunk(1, is_b, id_b, sem_ib)
        wait_chunk(is_a, id_a, sem_ia)
        for m in range(1, 4):
            pltpu.async_copy(zbuf.at[pl.ds(0, BATCH), :],
                             accum.at[id_a.at[0]], ssem[m], add=True)
        start_gather(is_a.at[0], 0)

        def slot(j, cur_is, cur_id, nxt_is):
            m, m1 = j % 4, (j + 1) % 4
            wait_scatter(m1)
            start_gather(nxt_is, m1)
            wait_gather(cur_is, m)
            start_scatter(m, cur_id)

        def _triple(q, _):
            for j in range(24):
                cur_is = is_x[j // G].at[j % G]
                cur_id = id_x[j // G].at[j % G]
                nj = j + 1
                if nj % G == 0:
                    k = (nj // G) % 3
                    wait_chunk(is_x[k], id_x[k], isem[k])
                nxt_is = is_x[(nj // G) % 3].at[nj % G]
                slot(j, cur_is, cur_id, nxt_is)
                if j == 3:
                    load_chunk(3 * q + 2, is_c, id_c, sem_ic)
                if j == 11:
                    load_chunk(3 * q + 3, is_a, id_a, sem_ia)
                if j == 19:
                    @pl.when(q < NT - 1)
                    def _():
                        load_chunk(3 * q + 4, is_b, id_b, sem_ib)
            return 0
        lax.fori_loop(0, NT, _triple, 0, unroll=False)

        for j in range(G - 1):
            slot(j, is_a.at[j], id_a.at[j], is_a.at[j + 1])
        m = (G - 1) % 4
        wait_gather(is_a.at[G - 1], m)
        start_scatter(m, id_a.at[G - 1])
        for m in range(4):
            wait_scatter(m)
        plsc.subcore_barrier()

        pltpu.sync_copy(accum.at[pl.ds(base, RPT), :],
                        out.at[pl.ds(base, RPT), :])
        plsc.subcore_barrier()

    @pl.when(c == 0)
    def _():
        do_block(t0, o0)

    @pl.when(c == 1)
    def _():
        do_block(t2, o2)


@functools.cache
def _agg():
    return pl.kernel(
        _agg_body,
        out_type=[jax.ShapeDtypeStruct((N2, CB), jnp.float32)
                  for _ in range(2)],
        mesh=_mesh(),
        compiler_params=pltpu.CompilerParams(use_tc_tiling_on_sc=False),
        scratch_types=[
            pltpu.VMEM((G, BATCH), jnp.int32),
            pltpu.VMEM((G, BATCH), jnp.int32),
            pltpu.VMEM((G, BATCH), jnp.int32),
            pltpu.VMEM((G, BATCH), jnp.int32),
            pltpu.VMEM((G, BATCH), jnp.int32),
            pltpu.VMEM((G, BATCH), jnp.int32),
            pltpu.VMEM((BATCH, CB), jnp.float32),
            pltpu.VMEM((BATCH, CB), jnp.float32),
            pltpu.VMEM((BATCH, CB), jnp.float32),
            pltpu.VMEM((BATCH, CB), jnp.float32),
            pltpu.VMEM((200, CB), jnp.float32),
            pltpu.VMEM_SHARED((N2, CB), jnp.float32),
        ] + [pltpu.SemaphoreType.DMA] * 11,
    )


def _cnt_body(dst_h, src_h, cnt_r, cnt_u, idx, ones, zbuf, accum, sem):
    c = lax.axis_index("c")
    s = lax.axis_index("s")

    def _ofill(j, _):
        ones[j, pl.ds(0, 16)] = jnp.ones((16,), jnp.float32)
        return 0
    lax.fori_loop(0, BATCH, _ofill, 0, unroll=False)

    def _zfill(j, _):
        zbuf[j, pl.ds(0, 16)] = jnp.zeros((16,), jnp.float32)
        return 0
    lax.fori_loop(0, zbuf.shape[0], _zfill, 0, unroll=False)

    def do_count(idx_h, out):
        row0 = pl.multiple_of(s * EPT, EPT)
        pltpu.sync_copy(idx_h.at[pl.ds(row0, EPT), :], idx)
        base = pl.multiple_of(s * RPT, RPT)
        _zero_accum_slice(zbuf, accum, base)
        plsc.subcore_barrier()

        def _q(iq, _):
            i0 = iq * 4
            for j in range(4):
                pltpu.async_copy(ones, accum.at[idx.at[i0 + j]], sem,
                                 add=True)
            for j in range(4):
                pltpu.make_async_copy(ones, accum.at[idx.at[i0 + j]],
                                      sem).wait()
            return 0
        lax.fori_loop(0, EPT // 4, _q, 0, unroll=False)
        plsc.subcore_barrier()

        pltpu.sync_copy(accum.at[pl.ds(base, RPT), :],
                        out.at[pl.ds(base, RPT), :])
        plsc.subcore_barrier()

    @pl.when(c == 0)
    def _():
        do_count(dst_h, cnt_r)

    @pl.when(c == 1)
    def _():
        do_count(src_h, cnt_u)


@functools.cache
def _cnt():
    return pl.kernel(
        _cnt_body,
        out_type=[jax.ShapeDtypeStruct((N2, 16), jnp.float32)
                  for _ in range(2)],
        mesh=_mesh(),
        compiler_params=pltpu.CompilerParams(use_tc_tiling_on_sc=False),
        scratch_types=[
            pltpu.VMEM((EPT, BATCH), jnp.int32),
            pltpu.VMEM((BATCH, 16), jnp.float32),
            pltpu.VMEM((200, 16), jnp.float32),
            pltpu.VMEM_SHARED((N2, 16), jnp.float32),
            pltpu.SemaphoreType.DMA,
        ],
    )


def _cls_body(u0, u1, u2, u3, r0, r1, r2, r3, eliu_h, elir_h,
              fu0, fu1, fu2, fu3, fr0, fr1, fr2, fr3,
              idxu, idxr, gb0, gb1, gb2, gb3, gb4, gb5, gb6, gb7,
              sem_g, sem_w):
    c = lax.axis_index("c")
    s = lax.axis_index("s")
    wid = s * NC + c
    tabs = (u0, u1, u2, u3, r0, r1, r2, r3)
    outs = (fu0, fu1, fu2, fu3, fr0, fr1, fr2, fr3)
    bufs = (gb0, gb1, gb2, gb3, gb4, gb5, gb6, gb7)

    e0 = pl.multiple_of(wid * LPT * BATCH, BATCH)
    pltpu.sync_copy(eliu_h.at[pl.ds(e0, LPT * BATCH)], idxu)
    pltpu.sync_copy(elir_h.at[pl.ds(e0, LPT * BATCH)], idxr)

    def _batch(i, _):
        iu = idxu.at[pl.ds(i * BATCH, BATCH)]
        ir = idxr.at[pl.ds(i * BATCH, BATCH)]
        idxs = (iu, iu, iu, iu, ir, ir, ir, ir)
        for k in range(8):
            pltpu.async_copy(tabs[k].at[idxs[k]], bufs[k], sem_g)
        off = e0 + i * BATCH
        for k in range(8):
            pltpu.make_async_copy(tabs[k].at[idxs[k]], bufs[k], sem_g).wait()
            pltpu.async_copy(bufs[k], outs[k].at[pl.ds(off, BATCH), :],
                             sem_w)
        for k in range(8):
            pltpu.make_async_copy(bufs[k], outs[k].at[pl.ds(off, BATCH), :],
                                  sem_w).wait()
        return 0
    lax.fori_loop(0, LPT, _batch, 0, unroll=False)


@functools.cache
def _cls():
    return pl.kernel(
        _cls_body,
        out_type=[jax.ShapeDtypeStruct((EL_PAD, CB), jnp.float32)
                  for _ in range(8)],
        mesh=_mesh(),
        compiler_params=pltpu.CompilerParams(use_tc_tiling_on_sc=False),
        scratch_types=(
            [pltpu.VMEM((LPT * BATCH,), jnp.int32) for _ in range(2)]
            + [pltpu.VMEM((BATCH, CB), jnp.float32) for _ in range(8)]
            + [pltpu.SemaphoreType.DMA, pltpu.SemaphoreType.DMA]
        ),
    )


def _dot_tc(fu0, fu1, fu2, fu3, fr0, fr1, fr2, fr3, o_ref):
    fu = jnp.concatenate([fu0[...], fu1[...], fu2[...], fu3[...]], axis=1)
    fr = jnp.concatenate([fr0[...], fr1[...], fr2[...], fr3[...]], axis=1)
    o_ref[...] = jnp.sum(fu * fr, axis=1, keepdims=True)


_RD = 2048


def _dot_call(feats):
    return pl.pallas_call(
        _dot_tc,
        grid=(EL_PAD // _RD,),
        in_specs=[pl.BlockSpec((_RD, CB), lambda i: (i, 0))
                  for _ in range(8)],
        out_specs=pl.BlockSpec((_RD, 1), lambda i: (i, 0)),
        out_shape=jax.ShapeDtypeStruct((EL_PAD, 1), jnp.float32),
    )(*feats)


_R = 1000


def _init_tc(x_ref, w_ref, b_ref, e_ref, o0, o1, o2, o3):
    x = jnp.dot(x_ref[...], w_ref[...],
                preferred_element_type=jnp.float32)
    x = x + b_ref[...] + e_ref[...]
    for b, o in enumerate((o0, o1, o2, o3)):
        o[...] = x[:, b * CB:(b + 1) * CB]


def _init_call(x, w, bias, emb):
    k = x.shape[1]
    return pl.pallas_call(
        _init_tc,
        grid=(N // _R,),
        in_specs=[
            pl.BlockSpec((_R, k), lambda i: (i, 0)),
            pl.BlockSpec((k, H), lambda i: (0, 0)),
            pl.BlockSpec((1, H), lambda i: (0, 0)),
            pl.BlockSpec((_R, H), lambda i: (i, 0)),
        ],
        out_specs=[pl.BlockSpec((_R, CB), lambda i: (i, 0))
                   for _ in range(NB)],
        out_shape=[jax.ShapeDtypeStruct((N2, CB), jnp.float32)
                   for _ in range(NB)],
    )(x, w, bias.reshape(1, H), emb)


def _sage_one(relu, ablks, c_ref, xblks, wl_ref, bl_ref, wr_ref):
    agg = jnp.concatenate([a[...] for a in ablks], axis=1)
    xd = jnp.concatenate([x[...] for x in xblks], axis=1)
    cnt = c_ref[:, 0:1]
    recip = 1.0 / jnp.maximum(cnt, 1.0)
    out = recip * jnp.dot(agg, wl_ref[...],
                          preferred_element_type=jnp.float32)
    out = out + bl_ref[...]
    out = out + jnp.dot(xd, wr_ref[...], preferred_element_type=jnp.float32)
    if relu:
        out = jnp.maximum(out, 0.0)
    return out


def _sage_call(agg, cnt, xdst, wl, bl, wr, relu):
    def body(*refs):
        outs = refs[12:]
        out = _sage_one(relu, refs[0:4], refs[4], refs[5:9], refs[9],
                        refs[10], refs[11])
        for b in range(NB):
            outs[b][...] = out[:, b * CB:(b + 1) * CB]

    blk = pl.BlockSpec((_R, CB), lambda i: (i, 0))
    cntblk = pl.BlockSpec((_R, 16), lambda i: (i, 0))
    wblk = pl.BlockSpec((H, H), lambda i: (0, 0))
    bblk = pl.BlockSpec((1, H), lambda i: (0, 0))
    return pl.pallas_call(
        body,
        grid=(N // _R,),
        in_specs=[blk] * 4 + [cntblk] + [blk] * 4 + [wblk, bblk, wblk],
        out_specs=[blk for _ in range(NB)],
        out_shape=[jax.ShapeDtypeStruct((N2, CB), jnp.float32)
                   for _ in range(NB)],
    )(*agg, cnt, *xdst, wl, bl.reshape(1, H), wr)


def kernel(x_user, x_restaurant, user_node_id, restaurant_node_id,
           edge_index, edge_label_index, W_user, b_user, W_rest, b_rest,
           emb_user, emb_rest,
           c1ur_Wl, c1ur_bl, c1ur_Wr, c1ru_Wl, c1ru_bl, c1ru_Wr,
           c2ur_Wl, c2ur_bl, c2ur_Wr, c2ru_Wl, c2ru_bl, c2ru_Wr):
    i32 = jnp.int32
    src, dst = edge_index[0], edge_index[1]
    zpad = jnp.zeros((E_PAD - E,), i32)
    dpad = jnp.full((E_PAD - E,), N, i32)
    src_g = jnp.concatenate([src, zpad]).reshape(EB, BATCH)
    src_s = jnp.concatenate([src, dpad]).reshape(EB, BATCH)
    dst_g = jnp.concatenate([dst, zpad]).reshape(EB, BATCH)
    dst_s = jnp.concatenate([dst, dpad]).reshape(EB, BATCH)
    lpad = jnp.zeros((EL_PAD - EL,), i32)
    eliu = jnp.concatenate([edge_label_index[0], lpad])
    elir = jnp.concatenate([edge_label_index[1], lpad])

    cnt_r, cnt_u = _cnt()(dst_s, src_s)

    xu = _init_call(x_user, W_user, b_user, emb_user)
    xr = _init_call(x_restaurant, W_rest, b_rest, emb_rest)

    def agg4(tabs, gat, sca):
        a02 = _agg()(tabs[0], tabs[2], gat, sca)
        a13 = _agg()(tabs[1], tabs[3], gat, sca)
        return [a02[0], a13[0], a02[1], a13[1]]

    agg_r = agg4(xu, src_g, dst_s)
    agg_u = agg4(xr, dst_g, src_s)
    xr1 = _sage_call(agg_r, cnt_r, xr, c1ur_Wl, c1ur_bl, c1ur_Wr, True)
    xu1 = _sage_call(agg_u, cnt_u, xu, c1ru_Wl, c1ru_bl, c1ru_Wr, True)
    agg_r = agg4(xu1, src_g, dst_s)
    agg_u = agg4(xr1, dst_g, src_s)
    xr2 = _sage_call(agg_r, cnt_r, xr1, c2ur_Wl, c2ur_bl, c2ur_Wr, True)
    xu2 = _sage_call(agg_u, cnt_u, xu1, c2ru_Wl, c2ru_bl, c2ru_Wr, True)
    agg_r = agg4(xu2, src_g, dst_s)
    agg_u = agg4(xr2, dst_g, src_s)
    xr3 = _sage_call(agg_r, cnt_r, xr2, c2ur_Wl, c2ur_bl, c2ur_Wr, False)
    xu3 = _sage_call(agg_u, cnt_u, xu2, c2ru_Wl, c2ru_bl, c2ru_Wr, False)

    feats = _cls()(*xu3, *xr3, eliu, elir)
    scores = _dot_call(feats)
    return scores[:EL, 0]


# speedup vs baseline: 1.2342x; 1.0010x over previous
import functools

import jax
import jax.numpy as jnp
from jax import lax
from jax.experimental import pallas as pl
from jax.experimental.pallas import tpu as pltpu
from jax.experimental.pallas import tpu_sc as plsc

N = 50000
N2 = 50048
RPT = N2 // 16
H = 128
CB = 32
NB = 4
NC = 2
NS = 16
BATCH = 128

E = 600000
EPT = 296
EB = EPT * NS
E_PAD = EB * BATCH

EL = 100000
LPT = 25
LB = LPT * NC * NS
EL_PAD = LB * BATCH


@functools.cache
def _mesh():
    return plsc.VectorSubcoreMesh(core_axis_name="c", subcore_axis_name="s",
                                  num_cores=NC, num_subcores=NS)


def _zero_accum_slice(zbuf, accum, base, sem):
    def descs():
        for k in range(15):
            yield pltpu.make_async_copy(
                zbuf, accum.at[pl.ds(base + k * 200, 200), :], sem)
        yield pltpu.make_async_copy(
            zbuf.at[pl.ds(0, 128), :],
            accum.at[pl.ds(base + 3000, 128), :], sem)
    for d in descs():
        d.start()
    for d in descs():
        d.wait()


G = 8
NCH = EPT // G
NT = 12


def _agg_body(t0, t2, gat_h, sca_h, o0, o2,
              is_a, is_b, is_c, id_a, id_b, id_c, r0, r1, r2, r3,
              zbuf, accum,
              g0, g1, g2, g3, s0, s1, s2, s3, sem_ia, sem_ib, sem_ic):
    c = lax.axis_index("c")
    s = lax.axis_index("s")
    row0 = pl.multiple_of(s * EPT, EPT)
    rows = (r0, r1, r2, r3)
    gsem = (g0, g1, g2, g3)
    ssem = (s0, s1, s2, s3)
    is_x = (is_a, is_b, is_c)
    id_x = (id_a, id_b, id_c)
    isem = (sem_ia, sem_ib, sem_ic)

    def _zfill(j, _):
        zbuf[j, pl.ds(0, 16)] = jnp.zeros((16,), jnp.float32)
        zbuf[j, pl.ds(16, 16)] = jnp.zeros((16,), jnp.float32)
        return 0
    lax.fori_loop(0, zbuf.shape[0], _zfill, 0, unroll=False)

    def do_block(tbl, out):
        def chunk_copies(ch, is_x, id_x, sem_x):
            r = pl.multiple_of(row0 + ch * G, G)
            return (
                pltpu.make_async_copy(gat_h.at[pl.ds(r, G), :], is_x, sem_x),
                pltpu.make_async_copy(sca_h.at[pl.ds(r, G), :], id_x, sem_x))

        def load_chunk(ch, is_x, id_x, sem_x):
            for d in chunk_copies(ch, is_x, id_x, sem_x):
                d.start()

        def wait_chunk(is_x, id_x, sem_x):
            for d in chunk_copies(0, is_x, id_x, sem_x):
                d.wait()
        base = pl.multiple_of(s * RPT, RPT)
        _zero_accum_slice(zbuf, accum, base, g0)
        plsc.subcore_barrier()

        def start_gather(idx_row, m):
            pltpu.async_copy(tbl.at[idx_row], rows[m], gsem[m])

        def wait_gather(idx_row, m):
            pltpu.make_async_copy(tbl.at[idx_row], rows[m], gsem[m]).wait()

        def start_scatter(m, id_row):
            pltpu.async_copy(rows[m], accum.at[id_row], ssem[m], add=True)

        def wait_scatter(m):
            pltpu.make_async_copy(rows[m], accum.at[id_a.at[0]],
                                  ssem[m]).wait()

        load_chunk(0, is_a, id_a, sem_ia)
        load_chunk(1, is_b, id_b, sem_ib)
        wait_chunk(is_a, id_a, sem_ia)
        for m in range(1, 4):
            pltpu.async_copy(zbuf.at[pl.ds(0, BATCH), :],
                             accum.at[id_a.at[0]], ssem[m], add=True)
        start_gather(is_a.at[0], 0)

        def slot(j, cur_is, cur_id, nxt_is):
            m, m1 = j % 4, (j + 1) % 4
            wait_scatter(m1)
            start_gather(nxt_is, m1)
            wait_gather(cur_is, m)
            start_scatter(m, cur_id)

        def _triple(q, _):
            for j in range(24):
                cur_is = is_x[j // G].at[j % G]
                cur_id = id_x[j // G].at[j % G]
                nj = j + 1
                if nj % G == 0:
                    k = (nj // G) % 3
                    wait_chunk(is_x[k], id_x[k], isem[k])
                nxt_is = is_x[(nj // G) % 3].at[nj % G]
                slot(j, cur_is, cur_id, nxt_is)
                if j == 3:
                    load_chunk(3 * q + 2, is_c, id_c, sem_ic)
                if j == 11:
                    load_chunk(3 * q + 3, is_a, id_a, sem_ia)
                if j == 19:
                    @pl.when(q < NT - 1)
                    def _():
                        load_chunk(3 * q + 4, is_b, id_b, sem_ib)
            return 0
        lax.fori_loop(0, NT, _triple, 0, unroll=False)

        for j in range(G - 1):
            slot(j, is_a.at[j], id_a.at[j], is_a.at[j + 1])
        m = (G - 1) % 4
        wait_gather(is_a.at[G - 1], m)
        start_scatter(m, id_a.at[G - 1])
        for m in range(4):
            wait_scatter(m)
        plsc.subcore_barrier()

        pltpu.sync_copy(accum.at[pl.ds(base, RPT), :],
                        out.at[pl.ds(base, RPT), :])
        plsc.subcore_barrier()

    @pl.when(c == 0)
    def _():
        do_block(t0, o0)

    @pl.when(c == 1)
    def _():
        do_block(t2, o2)


@functools.cache
def _agg():
    return pl.kernel(
        _agg_body,
        out_type=[jax.ShapeDtypeStruct((N2, CB), jnp.float32)
                  for _ in range(2)],
        mesh=_mesh(),
        compiler_params=pltpu.CompilerParams(use_tc_tiling_on_sc=False),
        scratch_types=[
            pltpu.VMEM((G, BATCH), jnp.int32),
            pltpu.VMEM((G, BATCH), jnp.int32),
            pltpu.VMEM((G, BATCH), jnp.int32),
            pltpu.VMEM((G, BATCH), jnp.int32),
            pltpu.VMEM((G, BATCH), jnp.int32),
            pltpu.VMEM((G, BATCH), jnp.int32),
            pltpu.VMEM((BATCH, CB), jnp.float32),
            pltpu.VMEM((BATCH, CB), jnp.float32),
            pltpu.VMEM((BATCH, CB), jnp.float32),
            pltpu.VMEM((BATCH, CB), jnp.float32),
            pltpu.VMEM((200, CB), jnp.float32),
            pltpu.VMEM_SHARED((N2, CB), jnp.float32),
        ] + [pltpu.SemaphoreType.DMA] * 11,
    )


def _cnt_body(dst_h, src_h, cnt_r, cnt_u, idx, ones, zbuf, accum, sem):
    c = lax.axis_index("c")
    s = lax.axis_index("s")

    def _ofill(j, _):
        ones[j, pl.ds(0, 16)] = jnp.ones((16,), jnp.float32)
        return 0
    lax.fori_loop(0, BATCH, _ofill, 0, unroll=False)

    def _zfill(j, _):
        zbuf[j, pl.ds(0, 16)] = jnp.zeros((16,), jnp.float32)
        return 0
    lax.fori_loop(0, zbuf.shape[0], _zfill, 0, unroll=False)

    def do_count(idx_h, out):
        row0 = pl.multiple_of(s * EPT, EPT)
        pltpu.sync_copy(idx_h.at[pl.ds(row0, EPT), :], idx)
        base = pl.multiple_of(s * RPT, RPT)
        _zero_accum_slice(zbuf, accum, base, sem)
        plsc.subcore_barrier()

        def _q(iq, _):
            i0 = iq * 4
            for j in range(4):
                pltpu.async_copy(ones, accum.at[idx.at[i0 + j]], sem,
                                 add=True)
            for j in range(4):
                pltpu.make_async_copy(ones, accum.at[idx.at[i0 + j]],
                                      sem).wait()
            return 0
        lax.fori_loop(0, EPT // 4, _q, 0, unroll=False)
        plsc.subcore_barrier()

        pltpu.sync_copy(accum.at[pl.ds(base, RPT), :],
                        out.at[pl.ds(base, RPT), :])
        plsc.subcore_barrier()

    @pl.when(c == 0)
    def _():
        do_count(dst_h, cnt_r)

    @pl.when(c == 1)
    def _():
        do_count(src_h, cnt_u)


@functools.cache
def _cnt():
    return pl.kernel(
        _cnt_body,
        out_type=[jax.ShapeDtypeStruct((N2, 16), jnp.float32)
                  for _ in range(2)],
        mesh=_mesh(),
        compiler_params=pltpu.CompilerParams(use_tc_tiling_on_sc=False),
        scratch_types=[
            pltpu.VMEM((EPT, BATCH), jnp.int32),
            pltpu.VMEM((BATCH, 16), jnp.float32),
            pltpu.VMEM((200, 16), jnp.float32),
            pltpu.VMEM_SHARED((N2, 16), jnp.float32),
            pltpu.SemaphoreType.DMA,
        ],
    )


def _cls_body(u0, u1, u2, u3, r0, r1, r2, r3, eliu_h, elir_h,
              fu0, fu1, fu2, fu3, fr0, fr1, fr2, fr3,
              idxu, idxr, gb0, gb1, gb2, gb3, gb4, gb5, gb6, gb7,
              sem_g, sem_w):
    c = lax.axis_index("c")
    s = lax.axis_index("s")
    wid = s * NC + c
    tabs = (u0, u1, u2, u3, r0, r1, r2, r3)
    outs = (fu0, fu1, fu2, fu3, fr0, fr1, fr2, fr3)
    bufs = (gb0, gb1, gb2, gb3, gb4, gb5, gb6, gb7)

    e0 = pl.multiple_of(wid * LPT * BATCH, BATCH)
    pltpu.sync_copy(eliu_h.at[pl.ds(e0, LPT * BATCH)], idxu)
    pltpu.sync_copy(elir_h.at[pl.ds(e0, LPT * BATCH)], idxr)

    def _batch(i, _):
        iu = idxu.at[pl.ds(i * BATCH, BATCH)]
        ir = idxr.at[pl.ds(i * BATCH, BATCH)]
        idxs = (iu, iu, iu, iu, ir, ir, ir, ir)
        for k in range(8):
            pltpu.async_copy(tabs[k].at[idxs[k]], bufs[k], sem_g)
        off = e0 + i * BATCH
        for k in range(8):
            pltpu.make_async_copy(tabs[k].at[idxs[k]], bufs[k], sem_g).wait()
            pltpu.async_copy(bufs[k], outs[k].at[pl.ds(off, BATCH), :],
                             sem_w)
        for k in range(8):
            pltpu.make_async_copy(bufs[k], outs[k].at[pl.ds(off, BATCH), :],
                                  sem_w).wait()
        return 0
    lax.fori_loop(0, LPT, _batch, 0, unroll=False)


@functools.cache
def _cls():
    return pl.kernel(
        _cls_body,
        out_type=[jax.ShapeDtypeStruct((EL_PAD, CB), jnp.float32)
                  for _ in range(8)],
        mesh=_mesh(),
        compiler_params=pltpu.CompilerParams(use_tc_tiling_on_sc=False),
        scratch_types=(
            [pltpu.VMEM((LPT * BATCH,), jnp.int32) for _ in range(2)]
            + [pltpu.VMEM((BATCH, CB), jnp.float32) for _ in range(8)]
            + [pltpu.SemaphoreType.DMA, pltpu.SemaphoreType.DMA]
        ),
    )


def _dot_tc(fu0, fu1, fu2, fu3, fr0, fr1, fr2, fr3, o_ref):
    fu = jnp.concatenate([fu0[...], fu1[...], fu2[...], fu3[...]], axis=1)
    fr = jnp.concatenate([fr0[...], fr1[...], fr2[...], fr3[...]], axis=1)
    o_ref[...] = jnp.sum(fu * fr, axis=1, keepdims=True)


_RD = 2048


def _dot_call(feats):
    return pl.pallas_call(
        _dot_tc,
        grid=(EL_PAD // _RD,),
        in_specs=[pl.BlockSpec((_RD, CB), lambda i: (i, 0))
                  for _ in range(8)],
        out_specs=pl.BlockSpec((_RD, 1), lambda i: (i, 0)),
        out_shape=jax.ShapeDtypeStruct((EL_PAD, 1), jnp.float32),
    )(*feats)


_R = 1000


def _init_tc(x_ref, w_ref, b_ref, e_ref, o0, o1, o2, o3):
    x = jnp.dot(x_ref[...], w_ref[...],
                preferred_element_type=jnp.float32)
    x = x + b_ref[...] + e_ref[...]
    for b, o in enumerate((o0, o1, o2, o3)):
        o[...] = x[:, b * CB:(b + 1) * CB]


def _init_call(x, w, bias, emb):
    k = x.shape[1]
    return pl.pallas_call(
        _init_tc,
        grid=(N // _R,),
        in_specs=[
            pl.BlockSpec((_R, k), lambda i: (i, 0)),
            pl.BlockSpec((k, H), lambda i: (0, 0)),
            pl.BlockSpec((1, H), lambda i: (0, 0)),
            pl.BlockSpec((_R, H), lambda i: (i, 0)),
        ],
        out_specs=[pl.BlockSpec((_R, CB), lambda i: (i, 0))
                   for _ in range(NB)],
        out_shape=[jax.ShapeDtypeStruct((N2, CB), jnp.float32)
                   for _ in range(NB)],
    )(x, w, bias.reshape(1, H), emb)


def _sage_one(relu, ablks, c_ref, xblks, wl_ref, bl_ref, wr_ref):
    agg = jnp.concatenate([a[...] for a in ablks], axis=1)
    xd = jnp.concatenate([x[...] for x in xblks], axis=1)
    cnt = c_ref[:, 0:1]
    recip = 1.0 / jnp.maximum(cnt, 1.0)
    out = recip * jnp.dot(agg, wl_ref[...],
                          preferred_element_type=jnp.float32)
    out = out + bl_ref[...]
    out = out + jnp.dot(xd, wr_ref[...], preferred_element_type=jnp.float32)
    if relu:
        out = jnp.maximum(out, 0.0)
    return out


def _sage_call(agg, cnt, xdst, wl, bl, wr, relu):
    def body(*refs):
        outs = refs[12:]
        out = _sage_one(relu, refs[0:4], refs[4], refs[5:9], refs[9],
                        refs[10], refs[11])
        for b in range(NB):
            outs[b][...] = out[:, b * CB:(b + 1) * CB]

    blk = pl.BlockSpec((_R, CB), lambda i: (i, 0))
    cntblk = pl.BlockSpec((_R, 16), lambda i: (i, 0))
    wblk = pl.BlockSpec((H, H), lambda i: (0, 0))
    bblk = pl.BlockSpec((1, H), lambda i: (0, 0))
    return pl.pallas_call(
        body,
        grid=(N // _R,),
        in_specs=[blk] * 4 + [cntblk] + [blk] * 4 + [wblk, bblk, wblk],
        out_specs=[blk for _ in range(NB)],
        out_shape=[jax.ShapeDtypeStruct((N2, CB), jnp.float32)
                   for _ in range(NB)],
    )(*agg, cnt, *xdst, wl, bl.reshape(1, H), wr)


def kernel(x_user, x_restaurant, user_node_id, restaurant_node_id,
           edge_index, edge_label_index, W_user, b_user, W_rest, b_rest,
           emb_user, emb_rest,
           c1ur_Wl, c1ur_bl, c1ur_Wr, c1ru_Wl, c1ru_bl, c1ru_Wr,
           c2ur_Wl, c2ur_bl, c2ur_Wr, c2ru_Wl, c2ru_bl, c2ru_Wr):
    i32 = jnp.int32
    src, dst = edge_index[0], edge_index[1]
    zpad = jnp.zeros((E_PAD - E,), i32)
    dpad = jnp.full((E_PAD - E,), N, i32)
    src_g = jnp.concatenate([src, zpad]).reshape(EB, BATCH)
    src_s = jnp.concatenate([src, dpad]).reshape(EB, BATCH)
    dst_g = jnp.concatenate([dst, zpad]).reshape(EB, BATCH)
    dst_s = jnp.concatenate([dst, dpad]).reshape(EB, BATCH)
    lpad = jnp.zeros((EL_PAD - EL,), i32)
    eliu = jnp.concatenate([edge_label_index[0], lpad])
    elir = jnp.concatenate([edge_label_index[1], lpad])

    cnt_r, cnt_u = _cnt()(dst_s, src_s)

    xu = _init_call(x_user, W_user, b_user, emb_user)
    xr = _init_call(x_restaurant, W_rest, b_rest, emb_rest)

    def agg4(tabs, gat, sca):
        a02 = _agg()(tabs[0], tabs[2], gat, sca)
        a13 = _agg()(tabs[1], tabs[3], gat, sca)
        return [a02[0], a13[0], a02[1], a13[1]]

    agg_r = agg4(xu, src_g, dst_s)
    agg_u = agg4(xr, dst_g, src_s)
    xr1 = _sage_call(agg_r, cnt_r, xr, c1ur_Wl, c1ur_bl, c1ur_Wr, True)
    xu1 = _sage_call(agg_u, cnt_u, xu, c1ru_Wl, c1ru_bl, c1ru_Wr, True)
    agg_r = agg4(xu1, src_g, dst_s)
    agg_u = agg4(xr1, dst_g, src_s)
    xr2 = _sage_call(agg_r, cnt_r, xr1, c2ur_Wl, c2ur_bl, c2ur_Wr, True)
    xu2 = _sage_call(agg_u, cnt_u, xu1, c2ru_Wl, c2ru_bl, c2ru_Wr, True)
    agg_r = agg4(xu2, src_g, dst_s)
    agg_u = agg4(xr2, dst_g, src_s)
    xr3 = _sage_call(agg_r, cnt_r, xr2, c2ur_Wl, c2ur_bl, c2ur_Wr, False)
    xu3 = _sage_call(agg_u, cnt_u, xu2, c2ru_Wl, c2ru_bl, c2ru_Wr, False)

    feats = _cls()(*xu3, *xr3, eliu, elir)
    scores = _dot_call(feats)
    return scores[:EL, 0]
